# Initial kernel scaffold; baseline (speedup 1.0000x reference)
import jax
import jax.numpy as jnp
from jax.experimental import pallas as pl


def kernel(positions, pe):
    raise NotImplementedError("write your pallas kernel here")


# speedup vs baseline: 1.9499x; 1.9499x over previous
import functools

import jax
import jax.numpy as jnp
from jax import lax
from jax.experimental import pallas as pl
from jax.experimental.pallas import tpu as pltpu
from jax.experimental.pallas import tpu_sc as plsc

DIM = 1024
NUM_WORKERS = 32
CHUNK = 64


def _gather_kernel_body(n_chunks, positions_hbm, pe_hbm, out_hbm,
                        idx_v, rows_v, sem):
    wid = lax.axis_index("s") * 2 + lax.axis_index("c")
    pltpu.sync_copy(positions_hbm.at[wid], idx_v)
    for c in range(n_chunks):
        row0 = (wid * n_chunks + c) * CHUNK
        pltpu.async_copy(pe_hbm.at[idx_v.at[c]], rows_v, sem).wait()
        pltpu.sync_copy(rows_v, out_hbm.at[pl.ds(row0, CHUNK)])


def kernel(positions, pe):
    batch, seq_len = positions.shape
    total = batch * seq_len
    assert total % (NUM_WORKERS * CHUNK) == 0
    n_chunks = total // (NUM_WORKERS * CHUNK)

    mesh = plsc.VectorSubcoreMesh(core_axis_name="c", subcore_axis_name="s")
    k = functools.partial(
        pl.kernel,
        mesh=mesh,
        out_type=jax.ShapeDtypeStruct((total, DIM), jnp.float32),
        scratch_types=[
            pltpu.VMEM((n_chunks, CHUNK), jnp.int32),
            pltpu.VMEM((CHUNK, DIM), jnp.float32),
            pltpu.SemaphoreType.DMA,
        ],
    )(functools.partial(_gather_kernel_body, n_chunks))

    flat_idx = positions.reshape(NUM_WORKERS, n_chunks, CHUNK)
    out = k(flat_idx, pe)
    return out.reshape(batch, seq_len, DIM)


# speedup vs baseline: 2.0989x; 1.0764x over previous
import functools

import jax
import jax.numpy as jnp
from jax import lax
from jax.experimental import pallas as pl
from jax.experimental.pallas import tpu as pltpu
from jax.experimental.pallas import tpu_sc as plsc

DIM = 1024
NUM_WORKERS = 32
CHUNK = 32
NBUF = 3


def _gather_kernel_body(n_chunks, positions_hbm, pe_hbm, out_hbm,
                        idx_v, bufs, gsems, wsems):
    wid = lax.axis_index("s") * 2 + lax.axis_index("c")
    pltpu.sync_copy(positions_hbm.at[wid], idx_v)

    def start_gather(c):
        return pltpu.async_copy(
            pe_hbm.at[idx_v.at[c]], bufs[c % NBUF], gsems[c % NBUF])

    def start_write(c):
        row0 = (wid * n_chunks + c) * CHUNK
        return pltpu.async_copy(
            bufs[c % NBUF], out_hbm.at[pl.ds(row0, CHUNK)], wsems[c % NBUF])

    gh = {}
    wh = {}
    for c in range(min(NBUF - 1, n_chunks)):
        gh[c] = start_gather(c)
    for c in range(n_chunks):
        nxt = c + NBUF - 1
        if nxt < n_chunks:
            if nxt - NBUF >= 0:
                wh.pop(nxt - NBUF).wait()
            gh[nxt] = start_gather(nxt)
        gh.pop(c).wait()
        wh[c] = start_write(c)
    for c in sorted(wh):
        wh.pop(c).wait()


def kernel(positions, pe):
    batch, seq_len = positions.shape
    total = batch * seq_len
    assert total % (NUM_WORKERS * CHUNK) == 0
    n_chunks = total // (NUM_WORKERS * CHUNK)

    mesh = plsc.VectorSubcoreMesh(core_axis_name="c", subcore_axis_name="s")
    k = functools.partial(
        pl.kernel,
        mesh=mesh,
        out_type=jax.ShapeDtypeStruct((total, DIM), jnp.float32),
        scratch_types=[
            pltpu.VMEM((n_chunks, CHUNK), jnp.int32),
            [pltpu.VMEM((CHUNK, DIM), jnp.float32) for _ in range(NBUF)],
            [pltpu.SemaphoreType.DMA for _ in range(NBUF)],
            [pltpu.SemaphoreType.DMA for _ in range(NBUF)],
        ],
    )(functools.partial(_gather_kernel_body, n_chunks))

    flat_idx = positions.reshape(NUM_WORKERS, n_chunks, CHUNK)
    out = k(flat_idx, pe)
    return out.reshape(batch, seq_len, DIM)


# speedup vs baseline: 2.1112x; 1.0059x over previous
import functools

import jax
import jax.numpy as jnp
from jax import lax
from jax.experimental import pallas as pl
from jax.experimental.pallas import tpu as pltpu
from jax.experimental.pallas import tpu_sc as plsc

DIM = 1024
NUM_WORKERS = 32
CHUNK = 16
NBUF = 6


def _gather_kernel_body(n_chunks, positions_hbm, pe_hbm, out_hbm,
                        idx_v, bufs, gsems, wsems):
    wid = lax.axis_index("s") * 2 + lax.axis_index("c")
    pltpu.sync_copy(positions_hbm.at[wid], idx_v)

    def start_gather(c):
        return pltpu.async_copy(
            pe_hbm.at[idx_v.at[c]], bufs[c % NBUF], gsems[c % NBUF])

    def start_write(c):
        row0 = (wid * n_chunks + c) * CHUNK
        return pltpu.async_copy(
            bufs[c % NBUF], out_hbm.at[pl.ds(row0, CHUNK)], wsems[c % NBUF])

    gh = {}
    wh = {}
    for c in range(min(NBUF - 1, n_chunks)):
        gh[c] = start_gather(c)
    for c in range(n_chunks):
        nxt = c + NBUF - 1
        if nxt < n_chunks:
            if nxt - NBUF >= 0:
                wh.pop(nxt - NBUF).wait()
            gh[nxt] = start_gather(nxt)
        gh.pop(c).wait()
        wh[c] = start_write(c)
    for c in sorted(wh):
        wh.pop(c).wait()


def kernel(positions, pe):
    batch, seq_len = positions.shape
    total = batch * seq_len
    assert total % (NUM_WORKERS * CHUNK) == 0
    n_chunks = total // (NUM_WORKERS * CHUNK)

    mesh = plsc.VectorSubcoreMesh(core_axis_name="c", subcore_axis_name="s")
    k = functools.partial(
        pl.kernel,
        mesh=mesh,
        out_type=jax.ShapeDtypeStruct((total, DIM), jnp.float32),
        scratch_types=[
            pltpu.VMEM((n_chunks, CHUNK), jnp.int32),
            [pltpu.VMEM((CHUNK, DIM), jnp.float32) for _ in range(NBUF)],
            [pltpu.SemaphoreType.DMA for _ in range(NBUF)],
            [pltpu.SemaphoreType.DMA for _ in range(NBUF)],
        ],
    )(functools.partial(_gather_kernel_body, n_chunks))

    flat_idx = positions.reshape(NUM_WORKERS, n_chunks, CHUNK)
    out = k(flat_idx, pe)
    return out.reshape(batch, seq_len, DIM)
